# baseline (speedup 1.0000x reference)
import dataclasses
import functools

import jax
import jax.numpy as jnp
from jax import lax
from jax.experimental import pallas as pl
from jax.experimental.pallas import tpu as pltpu
from jax.experimental.pallas import tpu_sc as plsc

NUM_CORES = 2
NUM_SUBCORES = 16
NUM_TILES = NUM_CORES * NUM_SUBCORES
LANES = 16
EB = 64

def _mesh():
    return plsc.VectorSubcoreMesh(core_axis_name="c", subcore_axis_name="s")


def _sc_params():
    cp = pltpu.CompilerParams()
    if "needs_layout_passes" in pltpu.CompilerParams.__dataclass_fields__:
        cp = dataclasses.replace(cp, needs_layout_passes=False)
    return cp


def _deg_partials(dst_t, zdeg, np_pad, ch):
    @functools.partial(
        pl.kernel, mesh=_mesh(),
        out_type=jax.ShapeDtypeStruct((NUM_TILES, np_pad), jnp.float32),
        scratch_types=[
            pltpu.VMEM((ch, EB), jnp.int32),
            pltpu.VMEM((np_pad,), jnp.float32),
        ],
        compiler_params=_sc_params(),
    )
    def k(dst_hbm, zdeg_hbm, out_hbm, dst_v, deg_v):
        cid = lax.axis_index("c")
        sid = lax.axis_index("s")
        wid = cid * NUM_SUBCORES + sid
        pltpu.sync_copy(dst_hbm.at[wid], dst_v)
        pltpu.sync_copy(zdeg_hbm, deg_v)
        ones = jnp.ones((LANES,), jnp.float32)

        @pl.loop(0, ch)
        def _(c):
            for j in range(EB // LANES):
                idx = dst_v[c, pl.ds(j * LANES, LANES)]
                plsc.addupdate_scatter(deg_v, [idx], ones)

        pltpu.sync_copy(deg_v, out_hbm.at[wid])

    return k(dst_t, zdeg)


def _edge_aggregate(hp, src_pk, dst_pk, zrows, np_pad, ch):
    rows_per_tile = np_pad // NUM_SUBCORES
    hw = EB // 2

    @functools.partial(
        pl.kernel, mesh=_mesh(),
        out_type=jax.ShapeDtypeStruct((NUM_CORES, np_pad, 128), jnp.float32),
        scratch_types=[
            pltpu.VMEM((ch * hw,), jnp.int32),
            pltpu.VMEM((ch * hw,), jnp.int32),
            pltpu.VMEM((4 * EB,), jnp.int32),
            pltpu.VMEM((4, EB), jnp.int32),
            pltpu.VMEM((EB, 128), jnp.float32),
            pltpu.VMEM((EB, 128), jnp.float32),
            pltpu.VMEM((EB, 128), jnp.float32),
            pltpu.VMEM((EB, 128), jnp.float32),
            pltpu.VMEM_SHARED((np_pad, 128), jnp.float32),
        ] + [pltpu.SemaphoreType.DMA] * 8,
    )
    def k(hp_hbm, spk_hbm, dpk_hbm, z_hbm, out_hbm, spk_v, dpk_v, sstg, dstg,
          r0, r1, r2, r3, acc_sh, g0, g1, g2, g3, s0, s1, s2, s3):
        rows = (r0, r1, r2, r3)
        gsem = (g0, g1, g2, g3)
        ssem = (s0, s1, s2, s3)
        cid = lax.axis_index("c")
        sid = lax.axis_index("s")
        wid = cid * NUM_SUBCORES + sid
        pltpu.sync_copy(spk_hbm.at[wid], spk_v)
        pltpu.sync_copy(dpk_hbm.at[wid], dpk_v)

        mask = jnp.int32(0xFFFF)

        def unpack(i, b):
            for j in range(EB // 32):
                v = spk_v[pl.ds(i * hw + j * 16, 16)]
                sstg[pl.ds(b * EB + j * 32, 16)] = v & mask
                sstg[pl.ds(b * EB + j * 32 + 16, 16)] = (
                    jax.lax.shift_right_logical(v, 16))
                w = dpk_v[pl.ds(i * hw + j * 16, 16)]
                dstg[b, pl.ds(j * 32, 16)] = w & mask
                dstg[b, pl.ds(j * 32 + 16, 16)] = (
                    jax.lax.shift_right_logical(w, 16))

        def g_copy(b):
            return pltpu.make_async_copy(
                hp_hbm.at[sstg.at[pl.ds(b * EB, EB)]], rows[b], gsem[b])

        def s_copy(b):
            return pltpu.make_async_copy(
                rows[b], acc_sh.at[dstg.at[b]], ssem[b])

        for b in range(4):
            unpack(b, b)
            g_copy(b).start()
        pltpu.sync_copy(z_hbm, acc_sh.at[pl.ds(sid * rows_per_tile, rows_per_tile)])
        plsc.subcore_barrier()
        g_copy(0).wait()
        s_copy(0).start(add=True)
        g_copy(1).wait()
        s_copy(1).start(add=True)

        @pl.loop(4, ch, step=4)
        def _(q):
            for b in range(4):
                i = q + b
                b2 = (b + 2) % 4
                s_copy(b).wait()
                unpack(i, b)
                g_copy(b).start()
                g_copy(b2).wait()
                s_copy(b2).start(add=True)

        g_copy(2).wait()
        s_copy(2).start(add=True)
        g_copy(3).wait()
        s_copy(3).start(add=True)
        for b in range(4):
            s_copy(b).wait()

        plsc.subcore_barrier()
        pltpu.sync_copy(
            acc_sh.at[pl.ds(sid * rows_per_tile, rows_per_tile)],
            out_hbm.at[cid, pl.ds(sid * rows_per_tile, rows_per_tile)],
        )

    return k(hp, src_pk, dst_pk, zrows)


_PREC = jax.lax.Precision.HIGHEST


def _degsum_body(p_ref, o_ref):
    s = jnp.sum(p_ref[...], axis=0, keepdims=True)
    o_ref[...] = jax.lax.rsqrt(s + 1.0)


def _stage1_body(x_ref, w_ref, dinv_ref, o_ref):
    h = jnp.dot(x_ref[...], w_ref[...], preferred_element_type=jnp.float32,
                precision=_PREC)
    o_ref[...] = h * dinv_ref[...]


def _mid_body(p0_ref, p1_ref, hp_ref, dinv_ref, b_ref, w_ref, o_ref):
    agg = p0_ref[...] + p1_ref[...] + hp_ref[...]
    h = jnp.maximum(agg * dinv_ref[...] + b_ref[...], 0.0)
    o_ref[...] = jnp.dot(h, w_ref[...], preferred_element_type=jnp.float32,
                         precision=_PREC) * dinv_ref[...]


def _final_body(p0_ref, p1_ref, hp_ref, dinv_ref, b_ref, batch_ref, wl_ref,
                bl_ref, o_ref):
    agg = p0_ref[...] + p1_ref[...] + hp_ref[...]
    h = jnp.maximum(agg * dinv_ref[...] + b_ref[...], 0.0)
    n = h.shape[0]
    g = o_ref.shape[0]
    gid = jax.lax.broadcasted_iota(jnp.int32, (g, n), 0)
    mask = (gid == batch_ref[...]).astype(jnp.float32)
    cnt = jnp.sum(mask, axis=1, keepdims=True)
    pooled = jnp.dot(mask, h, preferred_element_type=jnp.float32,
                     precision=_PREC) / jnp.maximum(cnt, 1.0)
    logits = jnp.dot(pooled, wl_ref[...], preferred_element_type=jnp.float32,
                     precision=_PREC) + bl_ref[...]
    m = jnp.max(logits, axis=1, keepdims=True)
    lse = jnp.log(jnp.sum(jnp.exp(logits - m), axis=1, keepdims=True)) + m
    o_ref[...] = logits - lse


def _tc(body, out_shape, *args):
    return pl.pallas_call(body, out_shape=out_shape)(*args)


def kernel(x, edge_index, batch, W1, b1, W2, b2, W3, b3, Wlin, blin):
    n, d = x.shape
    h_dim = W1.shape[1]
    g = 64
    c_dim = Wlin.shape[1]
    e = edge_index.shape[1]

    rows_per_tile = -(-(n + 1) // NUM_SUBCORES)
    rows_per_tile = -(-rows_per_tile // 8) * 8
    np_pad = rows_per_tile * NUM_SUBCORES

    ch = -(-e // (NUM_TILES * EB))
    ch = -(-ch // 4) * 4
    e_pad = NUM_TILES * ch * EB
    src = edge_index[0].astype(jnp.int32)
    dst = edge_index[1].astype(jnp.int32)
    pad = e_pad - e
    src_f = jnp.concatenate([src, jnp.zeros((pad,), jnp.int32)])
    dst_f = jnp.concatenate([dst, jnp.full((pad,), n, jnp.int32)])
    dst_t = dst_f.reshape(NUM_TILES, ch, EB)
    sp = src_f.reshape(NUM_TILES, -1, 2)
    src_pk = sp[..., 0] | (sp[..., 1] << 16)
    dp = dst_f.reshape(NUM_TILES, -1, 2)
    dst_pk = dp[..., 0] | (dp[..., 1] << 16)

    zdeg = jnp.zeros((np_pad,), jnp.float32)
    zrows = jnp.zeros((rows_per_tile, h_dim), jnp.float32)

    deg_parts = _deg_partials(dst_t, zdeg, np_pad, ch)
    dinv_row = _tc(_degsum_body,
                   jax.ShapeDtypeStruct((1, np_pad), jnp.float32), deg_parts)
    dinv_col = dinv_row.reshape(np_pad, 1)[:n]

    b1r = b1.reshape(1, h_dim)
    b2r = b2.reshape(1, h_dim)
    b3r = b3.reshape(1, h_dim)
    blr = blin.reshape(1, c_dim)
    batch_row = batch.astype(jnp.int32).reshape(1, n)

    hp = _tc(_stage1_body, jax.ShapeDtypeStruct((n, h_dim), jnp.float32),
             x, W1, dinv_col)

    for (b_r, w_next) in ((b1r, W2), (b2r, W3)):
        parts = _edge_aggregate(hp, src_pk, dst_pk, zrows, np_pad, ch)
        hp = _tc(_mid_body, jax.ShapeDtypeStruct((n, h_dim), jnp.float32),
                 parts[0, :n], parts[1, :n], hp, dinv_col, b_r, w_next)

    parts = _edge_aggregate(hp, src_pk, dst_pk, zrows, np_pad, ch)
    out = _tc(_final_body, jax.ShapeDtypeStruct((g, c_dim), jnp.float32),
              parts[0, :n], parts[1, :n], hp, dinv_col, b3r, batch_row,
              Wlin, blr)
    return out


# speedup vs baseline: 1.4475x; 1.4475x over previous
import dataclasses
import functools

import jax
import jax.numpy as jnp
from jax import lax
from jax.experimental import pallas as pl
from jax.experimental.pallas import tpu as pltpu
from jax.experimental.pallas import tpu_sc as plsc

NUM_CORES = 2
NUM_SUBCORES = 16
NUM_TILES = NUM_CORES * NUM_SUBCORES
LANES = 16
EB = 128

def _mesh():
    return plsc.VectorSubcoreMesh(core_axis_name="c", subcore_axis_name="s")


def _sc_params():
    cp = pltpu.CompilerParams()
    if "needs_layout_passes" in pltpu.CompilerParams.__dataclass_fields__:
        cp = dataclasses.replace(cp, needs_layout_passes=False)
    return cp


def _deg_partials(dpk, zdeg, np_pad, half):
    @functools.partial(
        pl.kernel, mesh=_mesh(),
        out_type=jax.ShapeDtypeStruct((NUM_TILES, np_pad), jnp.float32),
        scratch_types=[
            pltpu.VMEM((half,), jnp.int32),
            pltpu.VMEM((np_pad,), jnp.float32),
        ],
        compiler_params=_sc_params(),
    )
    def k(dpk_hbm, zdeg_hbm, out_hbm, dpk_v, deg_v):
        cid = lax.axis_index("c")
        sid = lax.axis_index("s")
        wid = cid * NUM_SUBCORES + sid
        pltpu.sync_copy(dpk_hbm.at[wid], dpk_v)
        pltpu.sync_copy(zdeg_hbm, deg_v)
        ones = jnp.ones((LANES,), jnp.float32)
        mask = jnp.int32(0xFFFF)

        @pl.loop(0, half // LANES)
        def _(i):
            v = dpk_v[pl.ds(i * LANES, LANES)]
            plsc.addupdate_scatter(deg_v, [v & mask], ones)
            plsc.addupdate_scatter(
                deg_v, [jax.lax.shift_right_logical(v, 16)], ones)

        pltpu.sync_copy(deg_v, out_hbm.at[wid])

    return k(dpk, zdeg)


def _edge_aggregate(hp, src_pk, dst_pk, zrows, np_pad, ch):
    rows_per_tile = np_pad // NUM_SUBCORES
    half = ch * EB // 2
    hc = ch // 2

    @functools.partial(
        pl.kernel, mesh=_mesh(),
        out_type=jax.ShapeDtypeStruct((NUM_CORES, np_pad, 128), jnp.float32),
        scratch_types=[
            pltpu.VMEM((half,), jnp.int32),
            pltpu.VMEM((half,), jnp.int32),
            pltpu.VMEM((2 * EB,), jnp.int32),
            pltpu.VMEM((2, EB), jnp.int32),
            pltpu.VMEM((EB, 128), jnp.float32),
            pltpu.VMEM((EB, 128), jnp.float32),
            pltpu.VMEM_SHARED((np_pad, 128), jnp.float32),
            pltpu.SemaphoreType.DMA,
            pltpu.SemaphoreType.DMA,
        ],
    )
    def k(hp_hbm, spk_hbm, dpk_hbm, z_hbm, out_hbm, spk_v, dpk_v, sstg, dstg,
          r0, r1, acc_sh, g0, g1):
        rows = (r0, r1)
        gsem = (g0, g1)
        cid = lax.axis_index("c")
        sid = lax.axis_index("s")
        wid = cid * NUM_SUBCORES + sid
        pltpu.sync_copy(spk_hbm.at[wid], spk_v)
        pltpu.sync_copy(dpk_hbm.at[wid], dpk_v)

        mask = jnp.int32(0xFFFF)

        def unpack(c, b):
            base = jnp.where(c < hc, c, c - hc) * EB
            lo = c < hc
            for j in range(EB // LANES):
                v = spk_v[pl.ds(base + j * LANES, LANES)]
                sstg[pl.ds(b * EB + j * LANES, LANES)] = jnp.where(
                    lo, v & mask, jax.lax.shift_right_logical(v, 16))
                w = dpk_v[pl.ds(base + j * LANES, LANES)]
                dstg[b, pl.ds(j * LANES, LANES)] = jnp.where(
                    lo, w & mask, jax.lax.shift_right_logical(w, 16))

        def g_copy(b):
            return pltpu.make_async_copy(
                hp_hbm.at[sstg.at[pl.ds(b * EB, EB)]], rows[b], gsem[b])

        unpack(0, 0)
        g_copy(0).start()
        unpack(1, 1)
        g_copy(1).start()
        pltpu.sync_copy(z_hbm, acc_sh.at[pl.ds(sid * rows_per_tile, rows_per_tile)])
        plsc.subcore_barrier()

        @pl.loop(0, ch, step=2)
        def _(c):
            for b in range(2):
                i = c + b
                g_copy(b).wait()
                pltpu.sync_copy(rows[b], acc_sh.at[dstg.at[b]], add=True)

                @pl.when(i + 2 < ch)
                def _():
                    unpack(i + 2, b)
                    g_copy(b).start()

        plsc.subcore_barrier()
        pltpu.sync_copy(
            acc_sh.at[pl.ds(sid * rows_per_tile, rows_per_tile)],
            out_hbm.at[cid, pl.ds(sid * rows_per_tile, rows_per_tile)],
        )

    return k(hp, src_pk, dst_pk, zrows)


_PREC = jax.lax.Precision.HIGHEST


def _pack_body(s_ref, d_ref, os_ref, od_ref):
    half = os_ref.shape[1]
    a = s_ref[...]
    os_ref[...] = a[:, :half] | (a[:, half:] << 16)
    b = d_ref[...]
    od_ref[...] = b[:, :half] | (b[:, half:] << 16)


def _degsum_body(p_ref, o_ref):
    s = jnp.sum(p_ref[...], axis=0, keepdims=True)
    o_ref[...] = jax.lax.rsqrt(s + 1.0)


def _stage1_body(x_ref, w_ref, dinv_ref, o_ref):
    h = jnp.dot(x_ref[...], w_ref[...], preferred_element_type=jnp.float32,
                precision=_PREC)
    o_ref[...] = h * dinv_ref[...]


def _mid_body(p0_ref, p1_ref, hp_ref, dinv_ref, b_ref, w_ref, o_ref):
    agg = p0_ref[...] + p1_ref[...] + hp_ref[...]
    h = jnp.maximum(agg * dinv_ref[...] + b_ref[...], 0.0)
    o_ref[...] = jnp.dot(h, w_ref[...], preferred_element_type=jnp.float32,
                         precision=_PREC) * dinv_ref[...]


def _final_body(p0_ref, p1_ref, hp_ref, dinv_ref, b_ref, batch_ref, wl_ref,
                bl_ref, o_ref):
    agg = p0_ref[...] + p1_ref[...] + hp_ref[...]
    h = jnp.maximum(agg * dinv_ref[...] + b_ref[...], 0.0)
    n = h.shape[0]
    g = o_ref.shape[0]
    gid = jax.lax.broadcasted_iota(jnp.int32, (g, n), 0)
    mask = (gid == batch_ref[...]).astype(jnp.float32)
    cnt = jnp.sum(mask, axis=1, keepdims=True)
    pooled = jnp.dot(mask, h, preferred_element_type=jnp.float32,
                     precision=_PREC) / jnp.maximum(cnt, 1.0)
    logits = jnp.dot(pooled, wl_ref[...], preferred_element_type=jnp.float32,
                     precision=_PREC) + bl_ref[...]
    m = jnp.max(logits, axis=1, keepdims=True)
    lse = jnp.log(jnp.sum(jnp.exp(logits - m), axis=1, keepdims=True)) + m
    o_ref[...] = logits - lse


def _tc(body, out_shape, *args):
    return pl.pallas_call(body, out_shape=out_shape)(*args)


def kernel(x, edge_index, batch, W1, b1, W2, b2, W3, b3, Wlin, blin):
    n, d = x.shape
    h_dim = W1.shape[1]
    g = 64
    c_dim = Wlin.shape[1]
    e = edge_index.shape[1]

    rows_per_tile = -(-(n + 1) // NUM_SUBCORES)
    rows_per_tile = -(-rows_per_tile // 8) * 8
    np_pad = rows_per_tile * NUM_SUBCORES

    ch = -(-e // (NUM_TILES * EB))
    ch = -(-ch // 2) * 2
    e_pad = NUM_TILES * ch * EB
    epw = ch * EB
    half = epw // 2
    src = edge_index[0].astype(jnp.int32)
    dst = edge_index[1].astype(jnp.int32)
    pad = e_pad - e
    src_f = jnp.concatenate([src, jnp.zeros((pad,), jnp.int32)]) \
        .reshape(NUM_TILES, epw)
    dst_f = jnp.concatenate([dst, jnp.full((pad,), n, jnp.int32)]) \
        .reshape(NUM_TILES, epw)
    src_pk, dst_pk = pl.pallas_call(
        _pack_body,
        out_shape=(jax.ShapeDtypeStruct((NUM_TILES, half), jnp.int32),
                   jax.ShapeDtypeStruct((NUM_TILES, half), jnp.int32)),
    )(src_f, dst_f)

    zdeg = jnp.zeros((np_pad,), jnp.float32)
    zrows = jnp.zeros((rows_per_tile, h_dim), jnp.float32)

    deg_parts = _deg_partials(dst_pk, zdeg, np_pad, half)
    dinv_row = _tc(_degsum_body,
                   jax.ShapeDtypeStruct((1, np_pad), jnp.float32), deg_parts)
    dinv_col = dinv_row.reshape(np_pad, 1)[:n]

    b1r = b1.reshape(1, h_dim)
    b2r = b2.reshape(1, h_dim)
    b3r = b3.reshape(1, h_dim)
    blr = blin.reshape(1, c_dim)
    batch_row = batch.astype(jnp.int32).reshape(1, n)

    hp = _tc(_stage1_body, jax.ShapeDtypeStruct((n, h_dim), jnp.float32),
             x, W1, dinv_col)

    for (b_r, w_next) in ((b1r, W2), (b2r, W3)):
        parts = _edge_aggregate(hp, src_pk, dst_pk, zrows, np_pad, ch)
        hp = _tc(_mid_body, jax.ShapeDtypeStruct((n, h_dim), jnp.float32),
                 parts[0, :n], parts[1, :n], hp, dinv_col, b_r, w_next)

    parts = _edge_aggregate(hp, src_pk, dst_pk, zrows, np_pad, ch)
    out = _tc(_final_body, jax.ShapeDtypeStruct((g, c_dim), jnp.float32),
              parts[0, :n], parts[1, :n], hp, dinv_col, b3r, batch_row,
              Wlin, blr)
    return out


# speedup vs baseline: 1.4537x; 1.0043x over previous
import dataclasses
import functools

import jax
import jax.numpy as jnp
from jax import lax
from jax.experimental import pallas as pl
from jax.experimental.pallas import tpu as pltpu
from jax.experimental.pallas import tpu_sc as plsc

NUM_CORES = 2
NUM_SUBCORES = 16
NUM_TILES = NUM_CORES * NUM_SUBCORES
LANES = 16
EB = 128

def _mesh():
    return plsc.VectorSubcoreMesh(core_axis_name="c", subcore_axis_name="s")


def _sc_params():
    cp = pltpu.CompilerParams()
    if "needs_layout_passes" in pltpu.CompilerParams.__dataclass_fields__:
        cp = dataclasses.replace(cp, needs_layout_passes=False)
    return cp


def _deg_partials(dpk, zdeg, np_pad, half):
    @functools.partial(
        pl.kernel, mesh=_mesh(),
        out_type=jax.ShapeDtypeStruct((NUM_TILES, np_pad), jnp.float32),
        scratch_types=[
            pltpu.VMEM((half,), jnp.int32),
            pltpu.VMEM((np_pad,), jnp.float32),
        ],
        compiler_params=_sc_params(),
    )
    def k(dpk_hbm, zdeg_hbm, out_hbm, dpk_v, deg_v):
        cid = lax.axis_index("c")
        sid = lax.axis_index("s")
        wid = cid * NUM_SUBCORES + sid
        pltpu.sync_copy(dpk_hbm.at[wid], dpk_v)
        pltpu.sync_copy(zdeg_hbm, deg_v)
        ones = jnp.ones((LANES,), jnp.float32)
        mask = jnp.int32(0xFFFF)

        @pl.loop(0, half // LANES)
        def _(i):
            v = dpk_v[pl.ds(i * LANES, LANES)]
            plsc.addupdate_scatter(deg_v, [v & mask], ones)
            plsc.addupdate_scatter(
                deg_v, [jax.lax.shift_right_logical(v, 16)], ones)

        pltpu.sync_copy(deg_v, out_hbm.at[wid])

    return k(dpk, zdeg)


def _edge_aggregate(hp, src_pk, dst_pk, zrows, np_pad, ch):
    rows_per_tile = np_pad // NUM_SUBCORES
    half = ch * EB // 2
    hc = ch // 2

    @functools.partial(
        pl.kernel, mesh=_mesh(),
        out_type=jax.ShapeDtypeStruct((NUM_CORES, np_pad, 128), jnp.float32),
        scratch_types=[
            pltpu.VMEM((half,), jnp.int32),
            pltpu.VMEM((half,), jnp.int32),
            pltpu.VMEM((2 * EB,), jnp.int32),
            pltpu.VMEM((2, EB), jnp.int32),
            pltpu.VMEM((EB, 128), jnp.float32),
            pltpu.VMEM((EB, 128), jnp.float32),
            pltpu.VMEM_SHARED((np_pad, 128), jnp.float32),
            pltpu.SemaphoreType.DMA,
            pltpu.SemaphoreType.DMA,
        ],
    )
    def k(hp_hbm, spk_hbm, dpk_hbm, z_hbm, out_hbm, spk_v, dpk_v, sstg, dstg,
          r0, r1, acc_sh, g0, g1):
        rows = (r0, r1)
        gsem = (g0, g1)
        cid = lax.axis_index("c")
        sid = lax.axis_index("s")
        wid = cid * NUM_SUBCORES + sid
        pltpu.sync_copy(spk_hbm.at[wid], spk_v)
        pltpu.sync_copy(dpk_hbm.at[wid], dpk_v)

        mask = jnp.int32(0xFFFF)

        def unpack(c, b):
            base = jnp.where(c < hc, c, c - hc) * EB
            lo = c < hc
            for j in range(EB // LANES):
                v = spk_v[pl.ds(base + j * LANES, LANES)]
                sstg[pl.ds(b * EB + j * LANES, LANES)] = jnp.where(
                    lo, v & mask, jax.lax.shift_right_logical(v, 16))
                w = dpk_v[pl.ds(base + j * LANES, LANES)]
                dstg[b, pl.ds(j * LANES, LANES)] = jnp.where(
                    lo, w & mask, jax.lax.shift_right_logical(w, 16))

        def g_copy(b):
            return pltpu.make_async_copy(
                hp_hbm.at[sstg.at[pl.ds(b * EB, EB)]], rows[b], gsem[b])

        unpack(0, 0)
        g_copy(0).start()
        unpack(1, 1)
        g_copy(1).start()
        pltpu.sync_copy(z_hbm, acc_sh.at[pl.ds(sid * rows_per_tile, rows_per_tile)])
        plsc.subcore_barrier()

        @pl.loop(0, ch, step=2)
        def _(c):
            for b in range(2):
                i = c + b
                g_copy(b).wait()
                pltpu.sync_copy(rows[b], acc_sh.at[dstg.at[b]], add=True)

                @pl.when(i + 2 < ch)
                def _():
                    unpack(i + 2, b)
                    g_copy(b).start()

        plsc.subcore_barrier()
        pltpu.sync_copy(
            acc_sh.at[pl.ds(sid * rows_per_tile, rows_per_tile)],
            out_hbm.at[cid, pl.ds(sid * rows_per_tile, rows_per_tile)],
        )

    return k(hp, src_pk, dst_pk, zrows)


_PREC = jax.lax.Precision.HIGHEST


def _pack_body(s_ref, d_ref, os_ref, od_ref):
    half = os_ref.shape[1]
    a = s_ref[...]
    os_ref[...] = a[:, :half] | (a[:, half:] << 16)
    b = d_ref[...]
    od_ref[...] = b[:, :half] | (b[:, half:] << 16)


def _degsum_body(p_ref, o_ref):
    s = jnp.sum(p_ref[...], axis=0, keepdims=True)
    o_ref[...] = jax.lax.rsqrt(s + 1.0)


def _stage1_body(x_ref, w_ref, dinv_ref, o_ref):
    h = jnp.dot(x_ref[...], w_ref[...], preferred_element_type=jnp.float32,
                precision=_PREC)
    o_ref[...] = h * dinv_ref[...]


def _mid_body(p0_ref, p1_ref, hp_ref, dinv_ref, b_ref, w_ref, o_ref):
    agg = p0_ref[...] + p1_ref[...] + hp_ref[...]
    h = jnp.maximum(agg * dinv_ref[...] + b_ref[...], 0.0)
    o_ref[...] = jnp.dot(h, w_ref[...], preferred_element_type=jnp.float32,
                         precision=_PREC) * dinv_ref[...]


def _final_body(p0_ref, p1_ref, hp_ref, dinv_ref, b_ref, batch_ref, wl_ref,
                bl_ref, o_ref):
    agg = p0_ref[...] + p1_ref[...] + hp_ref[...]
    h = jnp.maximum(agg * dinv_ref[...] + b_ref[...], 0.0)
    n = h.shape[0]
    g = o_ref.shape[0]
    gid = jax.lax.broadcasted_iota(jnp.int32, (g, n), 0)
    mask = (gid == batch_ref[...]).astype(jnp.float32)
    cnt = jnp.sum(mask, axis=1, keepdims=True)
    pooled = jnp.dot(mask, h, preferred_element_type=jnp.float32,
                     precision=_PREC) / jnp.maximum(cnt, 1.0)
    logits = jnp.dot(pooled, wl_ref[...], preferred_element_type=jnp.float32,
                     precision=_PREC) + bl_ref[...]
    m = jnp.max(logits, axis=1, keepdims=True)
    lse = jnp.log(jnp.sum(jnp.exp(logits - m), axis=1, keepdims=True)) + m
    o_ref[...] = logits - lse


def _tc(body, out_shape, *args):
    return pl.pallas_call(body, out_shape=out_shape)(*args)


def kernel(x, edge_index, batch, W1, b1, W2, b2, W3, b3, Wlin, blin):
    n, d = x.shape
    h_dim = W1.shape[1]
    g = 64
    c_dim = Wlin.shape[1]
    e = edge_index.shape[1]

    rows_per_tile = -(-(n + 1) // NUM_SUBCORES)
    rows_per_tile = -(-rows_per_tile // 8) * 8
    np_pad = rows_per_tile * NUM_SUBCORES

    ch = -(-e // (NUM_TILES * EB))
    ch = -(-ch // 2) * 2
    e_pad = NUM_TILES * ch * EB
    epw = ch * EB
    half = epw // 2
    src = edge_index[0].astype(jnp.int32)
    dst = edge_index[1].astype(jnp.int32)
    pad = e_pad - e
    trash = n + (jnp.arange(pad, dtype=jnp.int32) % (np_pad - n))
    src_f = jnp.concatenate([src, jnp.zeros((pad,), jnp.int32)]) \
        .reshape(NUM_TILES, epw)
    dst_f = jnp.concatenate([dst, trash]).reshape(NUM_TILES, epw)
    src_pk, dst_pk = pl.pallas_call(
        _pack_body,
        out_shape=(jax.ShapeDtypeStruct((NUM_TILES, half), jnp.int32),
                   jax.ShapeDtypeStruct((NUM_TILES, half), jnp.int32)),
    )(src_f, dst_f)

    zdeg = jnp.zeros((np_pad,), jnp.float32)
    zrows = jnp.zeros((rows_per_tile, h_dim), jnp.float32)

    deg_parts = _deg_partials(dst_pk, zdeg, np_pad, half)
    dinv_row = _tc(_degsum_body,
                   jax.ShapeDtypeStruct((1, np_pad), jnp.float32), deg_parts)
    dinv_col = dinv_row.reshape(np_pad, 1)[:n]

    b1r = b1.reshape(1, h_dim)
    b2r = b2.reshape(1, h_dim)
    b3r = b3.reshape(1, h_dim)
    blr = blin.reshape(1, c_dim)
    batch_row = batch.astype(jnp.int32).reshape(1, n)

    hp = _tc(_stage1_body, jax.ShapeDtypeStruct((n, h_dim), jnp.float32),
             x, W1, dinv_col)

    for (b_r, w_next) in ((b1r, W2), (b2r, W3)):
        parts = _edge_aggregate(hp, src_pk, dst_pk, zrows, np_pad, ch)
        hp = _tc(_mid_body, jax.ShapeDtypeStruct((n, h_dim), jnp.float32),
                 parts[0, :n], parts[1, :n], hp, dinv_col, b_r, w_next)

    parts = _edge_aggregate(hp, src_pk, dst_pk, zrows, np_pad, ch)
    out = _tc(_final_body, jax.ShapeDtypeStruct((g, c_dim), jnp.float32),
              parts[0, :n], parts[1, :n], hp, dinv_col, b3r, batch_row,
              Wlin, blr)
    return out


# speedup vs baseline: 4.4647x; 3.0713x over previous
import dataclasses
import functools

import jax
import jax.numpy as jnp
from jax import lax
from jax.experimental import pallas as pl
from jax.experimental.pallas import tpu as pltpu
from jax.experimental.pallas import tpu_sc as plsc

NUM_CORES = 2
NUM_SUBCORES = 16
NUM_TILES = NUM_CORES * NUM_SUBCORES
LANES = 16
EB = 128

def _mesh():
    return plsc.VectorSubcoreMesh(core_axis_name="c", subcore_axis_name="s")


def _sc_params():
    cp = pltpu.CompilerParams()
    if "needs_layout_passes" in pltpu.CompilerParams.__dataclass_fields__:
        cp = dataclasses.replace(cp, needs_layout_passes=False)
    return cp


def _deg_partials(dpk, zdeg, np_pad, half):
    @functools.partial(
        pl.kernel, mesh=_mesh(),
        out_type=jax.ShapeDtypeStruct((NUM_TILES, np_pad), jnp.float32),
        scratch_types=[
            pltpu.VMEM((half,), jnp.int32),
            pltpu.VMEM((np_pad,), jnp.float32),
        ],
        compiler_params=_sc_params(),
    )
    def k(dpk_hbm, zdeg_hbm, out_hbm, dpk_v, deg_v):
        cid = lax.axis_index("c")
        sid = lax.axis_index("s")
        wid = cid * NUM_SUBCORES + sid
        pltpu.sync_copy(dpk_hbm.at[wid], dpk_v)
        pltpu.sync_copy(zdeg_hbm, deg_v)
        ones = jnp.ones((LANES,), jnp.float32)
        mask = jnp.int32(0xFFFF)

        @pl.loop(0, half // LANES)
        def _(i):
            v = dpk_v[pl.ds(i * LANES, LANES)]
            plsc.addupdate_scatter(deg_v, [v & mask], ones)
            plsc.addupdate_scatter(
                deg_v, [jax.lax.shift_right_logical(v, 16)], ones)

        pltpu.sync_copy(deg_v, out_hbm.at[wid])

    return k(dpk, zdeg)


def _edge_aggregate(hp, src_pk, dst_pk, zrows, np_pad, ch):
    rows_per_tile = np_pad // NUM_SUBCORES
    half = ch * EB // 2
    hc = ch // 2

    @functools.partial(
        pl.kernel, mesh=_mesh(),
        out_type=jax.ShapeDtypeStruct((NUM_CORES, np_pad, 128), jnp.float32),
        scratch_types=[
            pltpu.VMEM((half,), jnp.int32),
            pltpu.VMEM((half,), jnp.int32),
            pltpu.VMEM((2 * EB,), jnp.int32),
            pltpu.VMEM((2, EB), jnp.int32),
            pltpu.VMEM((EB, 128), jnp.float32),
            pltpu.VMEM((EB, 128), jnp.float32),
            pltpu.VMEM_SHARED((np_pad, 128), jnp.float32),
            pltpu.SemaphoreType.DMA,
            pltpu.SemaphoreType.DMA,
        ],
    )
    def k(hp_hbm, spk_hbm, dpk_hbm, z_hbm, out_hbm, spk_v, dpk_v, sstg, dstg,
          r0, r1, acc_sh, g0, g1):
        rows = (r0, r1)
        gsem = (g0, g1)
        cid = lax.axis_index("c")
        sid = lax.axis_index("s")
        wid = cid * NUM_SUBCORES + sid
        pltpu.sync_copy(spk_hbm.at[wid], spk_v)
        pltpu.sync_copy(dpk_hbm.at[wid], dpk_v)

        mask = jnp.int32(0xFFFF)

        def unpack(c, b):
            base = jnp.where(c < hc, c, c - hc) * EB
            lo = c < hc
            for j in range(EB // LANES):
                v = spk_v[pl.ds(base + j * LANES, LANES)]
                sstg[pl.ds(b * EB + j * LANES, LANES)] = jnp.where(
                    lo, v & mask, jax.lax.shift_right_logical(v, 16))
                w = dpk_v[pl.ds(base + j * LANES, LANES)]
                dstg[b, pl.ds(j * LANES, LANES)] = jnp.where(
                    lo, w & mask, jax.lax.shift_right_logical(w, 16))

        def g_copy(b):
            return pltpu.make_async_copy(
                hp_hbm.at[sstg.at[pl.ds(b * EB, EB)]], rows[b], gsem[b])

        unpack(0, 0)
        g_copy(0).start()
        unpack(1, 1)
        g_copy(1).start()
        pltpu.sync_copy(z_hbm, acc_sh.at[pl.ds(sid * rows_per_tile, rows_per_tile)])
        plsc.subcore_barrier()

        @pl.loop(0, ch, step=2)
        def _(c):
            for b in range(2):
                i = c + b
                g_copy(b).wait()
                pltpu.sync_copy(rows[b], acc_sh.at[dstg.at[b]], add=True)

                @pl.when(i + 2 < ch)
                def _():
                    unpack(i + 2, b)
                    g_copy(b).start()

        plsc.subcore_barrier()
        pltpu.sync_copy(
            acc_sh.at[pl.ds(sid * rows_per_tile, rows_per_tile)],
            out_hbm.at[cid, pl.ds(sid * rows_per_tile, rows_per_tile)],
        )

    return k(hp, src_pk, dst_pk, zrows)


_PREC = jax.lax.Precision.HIGHEST


def _pack_body(s_ref, d_ref, os_ref, od_ref):
    half = os_ref.shape[1]
    a = s_ref[...]
    os_ref[...] = a[:, :half] | (a[:, half:] << 16)
    b = d_ref[...]
    od_ref[...] = b[:, :half] | (b[:, half:] << 16)


def _degsum_body(p_ref, o_ref):
    s = jnp.sum(p_ref[...], axis=0, keepdims=True)
    o_ref[...] = jax.lax.rsqrt(s + 1.0)


def _stage1_body(x_ref, w_ref, dinv_ref, o_ref):
    h = jnp.dot(x_ref[...], w_ref[...], preferred_element_type=jnp.float32,
                precision=_PREC)
    o_ref[...] = h * dinv_ref[...]


def _mid_body(p0_ref, p1_ref, hp_ref, dinv_ref, b_ref, w_ref, o_ref):
    agg = p0_ref[...] + p1_ref[...] + hp_ref[...]
    h = jnp.maximum(agg * dinv_ref[...] + b_ref[...], 0.0)
    o_ref[...] = jnp.dot(h, w_ref[...], preferred_element_type=jnp.float32,
                         precision=_PREC) * dinv_ref[...]


def _final_body(p0_ref, p1_ref, hp_ref, dinv_ref, b_ref, batch_ref, wl_ref,
                bl_ref, o_ref):
    agg = p0_ref[...] + p1_ref[...] + hp_ref[...]
    h = jnp.maximum(agg * dinv_ref[...] + b_ref[...], 0.0)
    n = h.shape[0]
    g = o_ref.shape[0]
    gid = jax.lax.broadcasted_iota(jnp.int32, (g, n), 0)
    mask = (gid == batch_ref[...]).astype(jnp.float32)
    cnt = jnp.sum(mask, axis=1, keepdims=True)
    pooled = jnp.dot(mask, h, preferred_element_type=jnp.float32,
                     precision=_PREC) / jnp.maximum(cnt, 1.0)
    logits = jnp.dot(pooled, wl_ref[...], preferred_element_type=jnp.float32,
                     precision=_PREC) + bl_ref[...]
    m = jnp.max(logits, axis=1, keepdims=True)
    lse = jnp.log(jnp.sum(jnp.exp(logits - m), axis=1, keepdims=True)) + m
    o_ref[...] = logits - lse


def _tc(body, out_shape, *args):
    return pl.pallas_call(body, out_shape=out_shape)(*args)


def kernel(x, edge_index, batch, W1, b1, W2, b2, W3, b3, Wlin, blin):
    n, d = x.shape
    h_dim = W1.shape[1]
    g = 64
    c_dim = Wlin.shape[1]
    e = edge_index.shape[1]

    rows_per_tile = -(-(n + 1) // NUM_SUBCORES)
    rows_per_tile = -(-rows_per_tile // 8) * 8
    np_pad = rows_per_tile * NUM_SUBCORES

    ch = -(-e // (NUM_TILES * EB))
    ch = -(-ch // 2) * 2
    e_pad = NUM_TILES * ch * EB
    epw = ch * EB
    half = epw // 2
    src = edge_index[0].astype(jnp.int32)
    dst = edge_index[1].astype(jnp.int32)
    pad = e_pad - e
    trash = n + (jnp.arange(pad, dtype=jnp.int32) % (np_pad - n))
    psrc = jnp.arange(pad, dtype=jnp.int32) % n
    src_f = jnp.concatenate([src, psrc]).reshape(NUM_TILES, epw)
    dst_f = jnp.concatenate([dst, trash]).reshape(NUM_TILES, epw)
    src_pk, dst_pk = pl.pallas_call(
        _pack_body,
        out_shape=(jax.ShapeDtypeStruct((NUM_TILES, half), jnp.int32),
                   jax.ShapeDtypeStruct((NUM_TILES, half), jnp.int32)),
    )(src_f, dst_f)

    zdeg = jnp.zeros((np_pad,), jnp.float32)
    zrows = jnp.zeros((rows_per_tile, h_dim), jnp.float32)

    deg_parts = _deg_partials(dst_pk, zdeg, np_pad, half)
    dinv_row = _tc(_degsum_body,
                   jax.ShapeDtypeStruct((1, np_pad), jnp.float32), deg_parts)
    dinv_col = dinv_row.reshape(np_pad, 1)[:n]

    b1r = b1.reshape(1, h_dim)
    b2r = b2.reshape(1, h_dim)
    b3r = b3.reshape(1, h_dim)
    blr = blin.reshape(1, c_dim)
    batch_row = batch.astype(jnp.int32).reshape(1, n)

    hp = _tc(_stage1_body, jax.ShapeDtypeStruct((n, h_dim), jnp.float32),
             x, W1, dinv_col)

    for (b_r, w_next) in ((b1r, W2), (b2r, W3)):
        parts = _edge_aggregate(hp, src_pk, dst_pk, zrows, np_pad, ch)
        hp = _tc(_mid_body, jax.ShapeDtypeStruct((n, h_dim), jnp.float32),
                 parts[0, :n], parts[1, :n], hp, dinv_col, b_r, w_next)

    parts = _edge_aggregate(hp, src_pk, dst_pk, zrows, np_pad, ch)
    out = _tc(_final_body, jax.ShapeDtypeStruct((g, c_dim), jnp.float32),
              parts[0, :n], parts[1, :n], hp, dinv_col, b3r, batch_row,
              Wlin, blr)
    return out


# speedup vs baseline: 4.6937x; 1.0513x over previous
import dataclasses
import functools

import jax
import jax.numpy as jnp
from jax import lax
from jax.experimental import pallas as pl
from jax.experimental.pallas import tpu as pltpu
from jax.experimental.pallas import tpu_sc as plsc

NUM_CORES = 2
NUM_SUBCORES = 16
NUM_TILES = NUM_CORES * NUM_SUBCORES
LANES = 16
EB = 128

def _mesh():
    return plsc.VectorSubcoreMesh(core_axis_name="c", subcore_axis_name="s")


def _sc_params():
    cp = pltpu.CompilerParams()
    if "needs_layout_passes" in pltpu.CompilerParams.__dataclass_fields__:
        cp = dataclasses.replace(cp, needs_layout_passes=False)
    return cp


def _deg_partials(dpk, zdeg, np_pad, half):
    @functools.partial(
        pl.kernel, mesh=_mesh(),
        out_type=jax.ShapeDtypeStruct((NUM_TILES, np_pad), jnp.float32),
        scratch_types=[
            pltpu.VMEM((half,), jnp.int32),
            pltpu.VMEM((np_pad,), jnp.float32),
        ],
        compiler_params=_sc_params(),
    )
    def k(dpk_hbm, zdeg_hbm, out_hbm, dpk_v, deg_v):
        cid = lax.axis_index("c")
        sid = lax.axis_index("s")
        wid = cid * NUM_SUBCORES + sid
        pltpu.sync_copy(dpk_hbm.at[wid], dpk_v)
        pltpu.sync_copy(zdeg_hbm, deg_v)
        ones = jnp.ones((LANES,), jnp.float32)
        mask = jnp.int32(0xFFFF)

        @pl.loop(0, half // LANES)
        def _(i):
            v = dpk_v[pl.ds(i * LANES, LANES)]
            plsc.addupdate_scatter(deg_v, [v & mask], ones)
            plsc.addupdate_scatter(
                deg_v, [jax.lax.shift_right_logical(v, 16)], ones)

        pltpu.sync_copy(deg_v, out_hbm.at[wid])

    return k(dpk, zdeg)


def _edge_aggregate(hp, src_pk, dst_pk, zrows, np_pad, ch):
    rows_per_tile = np_pad // NUM_SUBCORES
    half = ch * EB // 2
    hc = ch // 2

    @functools.partial(
        pl.kernel, mesh=_mesh(),
        out_type=jax.ShapeDtypeStruct((NUM_CORES, np_pad, 128), jnp.float32),
        scratch_types=[
            pltpu.VMEM((half,), jnp.int32),
            pltpu.VMEM((half,), jnp.int32),
            pltpu.VMEM((2 * EB,), jnp.int32),
            pltpu.VMEM((2, EB), jnp.int32),
            pltpu.VMEM((EB, 128), jnp.float32),
            pltpu.VMEM((EB, 128), jnp.float32),
            pltpu.VMEM_SHARED((np_pad, 128), jnp.float32),
            pltpu.SemaphoreType.DMA,
            pltpu.SemaphoreType.DMA,
        ],
    )
    def k(hp_hbm, spk_hbm, dpk_hbm, z_hbm, out_hbm, spk_v, dpk_v, sstg, dstg,
          r0, r1, acc_sh, g0, g1):
        rows = (r0, r1)
        gsem = (g0, g1)
        cid = lax.axis_index("c")
        sid = lax.axis_index("s")
        wid = cid * NUM_SUBCORES + sid
        pltpu.sync_copy(spk_hbm.at[wid], spk_v)
        pltpu.sync_copy(dpk_hbm.at[wid], dpk_v)

        mask = jnp.int32(0xFFFF)

        def unpack(c, b):
            base = jnp.where(c < hc, c, c - hc) * EB
            lo = c < hc
            for j in range(EB // LANES):
                v = spk_v[pl.ds(base + j * LANES, LANES)]
                sstg[pl.ds(b * EB + j * LANES, LANES)] = jnp.where(
                    lo, v & mask, jax.lax.shift_right_logical(v, 16))
                w = dpk_v[pl.ds(base + j * LANES, LANES)]
                dstg[b, pl.ds(j * LANES, LANES)] = jnp.where(
                    lo, w & mask, jax.lax.shift_right_logical(w, 16))

        def g_copy(b):
            return pltpu.make_async_copy(
                hp_hbm.at[sstg.at[pl.ds(b * EB, EB)]], rows[b], gsem[b])

        unpack(0, 0)
        g_copy(0).start()
        unpack(1, 1)
        g_copy(1).start()
        pltpu.sync_copy(z_hbm, acc_sh.at[pl.ds(sid * rows_per_tile, rows_per_tile)])
        plsc.subcore_barrier()

        @pl.loop(0, ch, step=2)
        def _(c):
            for b in range(2):
                i = c + b
                g_copy(b).wait()
                pltpu.sync_copy(rows[b], acc_sh.at[dstg.at[b]], add=True)

                @pl.when(i + 2 < ch)
                def _():
                    unpack(i + 2, b)
                    g_copy(b).start()

        plsc.subcore_barrier()
        pltpu.sync_copy(
            acc_sh.at[pl.ds(sid * rows_per_tile, rows_per_tile)],
            out_hbm.at[cid, pl.ds(sid * rows_per_tile, rows_per_tile)],
        )

    return k(hp, src_pk, dst_pk, zrows)


_PREC = jax.lax.Precision.HIGHEST


def _pack_body(s_ref, d_ref, os_ref, od_ref):
    half = os_ref.shape[1]
    a = s_ref[...]
    os_ref[...] = a[:, :half] | (a[:, half:] << 16)
    b = d_ref[...]
    od_ref[...] = b[:, :half] | (b[:, half:] << 16)


def _degsum_body(p_ref, o_ref):
    s = jnp.sum(p_ref[...], axis=0, keepdims=True)
    o_ref[...] = jax.lax.rsqrt(s + 1.0)


def _stage1_body(x_ref, w_ref, dinv_ref, o_ref):
    h = jnp.dot(x_ref[...], w_ref[...], preferred_element_type=jnp.float32,
                precision=_PREC)
    o_ref[...] = h * dinv_ref[...]


def _mid_body(p_ref, hp_ref, dinv_ref, b_ref, w_ref, o_ref):
    n = hp_ref.shape[0]
    agg = p_ref[0, :n, :] + p_ref[1, :n, :] + hp_ref[...]
    h = jnp.maximum(agg * dinv_ref[...] + b_ref[...], 0.0)
    o_ref[...] = jnp.dot(h, w_ref[...], preferred_element_type=jnp.float32,
                         precision=_PREC) * dinv_ref[...]


def _final_body(p_ref, hp_ref, dinv_ref, b_ref, batch_ref, wl_ref,
                bl_ref, o_ref):
    n = hp_ref.shape[0]
    agg = p_ref[0, :n, :] + p_ref[1, :n, :] + hp_ref[...]
    h = jnp.maximum(agg * dinv_ref[...] + b_ref[...], 0.0)
    n = h.shape[0]
    g = o_ref.shape[0]
    gid = jax.lax.broadcasted_iota(jnp.int32, (g, n), 0)
    mask = (gid == batch_ref[...]).astype(jnp.float32)
    cnt = jnp.sum(mask, axis=1, keepdims=True)
    pooled = jnp.dot(mask, h, preferred_element_type=jnp.float32,
                     precision=_PREC) / jnp.maximum(cnt, 1.0)
    logits = jnp.dot(pooled, wl_ref[...], preferred_element_type=jnp.float32,
                     precision=_PREC) + bl_ref[...]
    m = jnp.max(logits, axis=1, keepdims=True)
    lse = jnp.log(jnp.sum(jnp.exp(logits - m), axis=1, keepdims=True)) + m
    o_ref[...] = logits - lse


def _tc(body, out_shape, *args):
    return pl.pallas_call(body, out_shape=out_shape)(*args)


def kernel(x, edge_index, batch, W1, b1, W2, b2, W3, b3, Wlin, blin):
    n, d = x.shape
    h_dim = W1.shape[1]
    g = 64
    c_dim = Wlin.shape[1]
    e = edge_index.shape[1]

    rows_per_tile = -(-(n + 1) // NUM_SUBCORES)
    rows_per_tile = -(-rows_per_tile // 8) * 8
    np_pad = rows_per_tile * NUM_SUBCORES

    ch = -(-e // (NUM_TILES * EB))
    ch = -(-ch // 2) * 2
    e_pad = NUM_TILES * ch * EB
    epw = ch * EB
    half = epw // 2
    src = edge_index[0].astype(jnp.int32)
    dst = edge_index[1].astype(jnp.int32)
    pad = e_pad - e
    trash = n + (jnp.arange(pad, dtype=jnp.int32) % (np_pad - n))
    psrc = jnp.arange(pad, dtype=jnp.int32) % n
    src_f = jnp.concatenate([src, psrc]).reshape(NUM_TILES, epw)
    dst_f = jnp.concatenate([dst, trash]).reshape(NUM_TILES, epw)
    src_pk, dst_pk = pl.pallas_call(
        _pack_body,
        out_shape=(jax.ShapeDtypeStruct((NUM_TILES, half), jnp.int32),
                   jax.ShapeDtypeStruct((NUM_TILES, half), jnp.int32)),
    )(src_f, dst_f)

    zdeg = jnp.zeros((np_pad,), jnp.float32)
    zrows = jnp.zeros((rows_per_tile, h_dim), jnp.float32)

    deg_parts = _deg_partials(dst_pk, zdeg, np_pad, half)
    dinv_row = _tc(_degsum_body,
                   jax.ShapeDtypeStruct((1, np_pad), jnp.float32), deg_parts)
    dinv_col = dinv_row.reshape(np_pad, 1)[:n]

    b1r = b1.reshape(1, h_dim)
    b2r = b2.reshape(1, h_dim)
    b3r = b3.reshape(1, h_dim)
    blr = blin.reshape(1, c_dim)
    batch_row = batch.astype(jnp.int32).reshape(1, n)

    hp = _tc(_stage1_body, jax.ShapeDtypeStruct((n, h_dim), jnp.float32),
             x, W1, dinv_col)

    for (b_r, w_next) in ((b1r, W2), (b2r, W3)):
        parts = _edge_aggregate(hp, src_pk, dst_pk, zrows, np_pad, ch)
        hp = _tc(_mid_body, jax.ShapeDtypeStruct((n, h_dim), jnp.float32),
                 parts, hp, dinv_col, b_r, w_next)

    parts = _edge_aggregate(hp, src_pk, dst_pk, zrows, np_pad, ch)
    out = _tc(_final_body, jax.ShapeDtypeStruct((g, c_dim), jnp.float32),
              parts, hp, dinv_col, b3r, batch_row, Wlin, blr)
    return out
